# baseline (speedup 1.0000x reference)
import functools

import jax
import jax.numpy as jnp
from jax import lax
from jax.experimental import pallas as pl
from jax.experimental.pallas import tpu as pltpu
from jax.experimental.pallas import tpu_sc as plsc

N_NODES = 10000
N_EDGES = 320000
D_IN = 128
D_H1 = 128
D_H2 = 16

NC, NS = 2, 16
NW = NC * NS
EPT = N_EDGES // NW
K = 80
KG = K // 16
NBLK = EPT // K
RCHUNK = 200
NCHUNK = N_NODES // RCHUNK
CHUNK_ITERS = -(-NCHUNK // NS)

_ROW_BLK = 1000
_NRB = N_NODES // _ROW_BLK

_SC_PARAMS = pltpu.CompilerParams(
    needs_layout_passes=False, use_tc_tiling_on_sc=False)


def _proj_body(x_ref, wq, bq, wk, bk, wv, bv, ws, bs,
               q_ref, k_ref, vlo_ref, vhi_ref, s_ref):
  xb = x_ref[...]
  q_ref[...] = jnp.dot(xb, wq[...], preferred_element_type=jnp.float32) + bq[...]
  k_ref[...] = jnp.dot(xb, wk[...], preferred_element_type=jnp.float32) + bk[...]
  v = jnp.dot(xb, wv[...], preferred_element_type=jnp.float32) + bv[...]
  half = v.shape[1] // 2
  vlo_ref[...] = v[:, :half]
  vhi_ref[...] = v[:, half:]
  s_ref[...] = jnp.dot(xb, ws[...], preferred_element_type=jnp.float32) + bs[...]


def _projections(x, wq, bq, wk, bk, wv, bv, ws, bs):
  n, d_in = x.shape
  d_out = wq.shape[1]
  half = d_out // 2
  row = pl.BlockSpec((_ROW_BLK, d_in), lambda i: (i, 0))
  wspec = pl.BlockSpec((d_in, d_out), lambda i: (0, 0))
  bspec = pl.BlockSpec((1, d_out), lambda i: (0, 0))
  ospec = pl.BlockSpec((_ROW_BLK, d_out), lambda i: (i, 0))
  hspec = pl.BlockSpec((_ROW_BLK, half), lambda i: (i, 0))
  out = jax.ShapeDtypeStruct((n, d_out), jnp.float32)
  outh = jax.ShapeDtypeStruct((n, half), jnp.float32)
  return pl.pallas_call(
      _proj_body,
      grid=(_NRB,),
      in_specs=[row, wspec, bspec, wspec, bspec, wspec, bspec, wspec, bspec],
      out_specs=[ospec, ospec, hspec, hspec, ospec],
      out_shape=[out, out, outh, outh, out],
  )(x, wq, bq.reshape(1, -1), wk, bk.reshape(1, -1),
    wv, bv.reshape(1, -1), ws, bs.reshape(1, -1))


def _combine_proj_body(pn_ref, pd_ref, s1_ref, wq, bq, wk, bk, wv, bv, ws, bs,
                       q_ref, k_ref, v_ref, s_ref):
  numer = jnp.concatenate([pn_ref[i] for i in range(NC)], axis=1)
  denom = pd_ref[0, :, 0:1]
  for i in range(1, NC):
    denom = denom + pd_ref[i, :, 0:1]
  h = jnp.maximum(numer / (denom + 1e-16) + s1_ref[...], 0.0)
  q_ref[...] = jnp.dot(h, wq[...], preferred_element_type=jnp.float32) + bq[...]
  k_ref[...] = jnp.dot(h, wk[...], preferred_element_type=jnp.float32) + bk[...]
  v_ref[...] = jnp.dot(h, wv[...], preferred_element_type=jnp.float32) + bv[...]
  s_ref[...] = jnp.dot(h, ws[...], preferred_element_type=jnp.float32) + bs[...]


def _combine_projections(pn, pd, s1, wq, bq, wk, bk, wv, bv, ws, bs):
  n, d_in = s1.shape
  d_out = wq.shape[1]
  pnspec = pl.BlockSpec((NC, _ROW_BLK, d_in // NC), lambda i: (0, i, 0))
  pdspec = pl.BlockSpec((NC, _ROW_BLK, 16), lambda i: (0, i, 0))
  sspec = pl.BlockSpec((_ROW_BLK, d_in), lambda i: (i, 0))
  wspec = pl.BlockSpec((d_in, d_out), lambda i: (0, 0))
  bspec = pl.BlockSpec((1, d_out), lambda i: (0, 0))
  ospec = pl.BlockSpec((_ROW_BLK, d_out), lambda i: (i, 0))
  out = jax.ShapeDtypeStruct((n, d_out), jnp.float32)
  return pl.pallas_call(
      _combine_proj_body,
      grid=(_NRB,),
      in_specs=[pnspec, pdspec, sspec, wspec, bspec, wspec, bspec,
                wspec, bspec, wspec, bspec],
      out_specs=[ospec, ospec, ospec, ospec],
      out_shape=[out, out, out, out],
  )(pn, pd, s1, wq, bq.reshape(1, -1), wk, bk.reshape(1, -1),
    wv, bv.reshape(1, -1), ws, bs.reshape(1, -1))


def _final_body(part_ref, s2_ref, out_ref):
  p = part_ref[0]
  for i in range(1, NC):
    p = p + part_ref[i]
  denom = p[:, 0:1]
  numer = p[:, 16:]
  out_ref[...] = numer / (denom + 1e-16) + s2_ref[...]


def _final_combine(part, s2):
  n, d_out = s2.shape
  w_in = part.shape[2]
  pspec = pl.BlockSpec((NC, _ROW_BLK, w_in), lambda i: (0, i, 0))
  sspec = pl.BlockSpec((_ROW_BLK, d_out), lambda i: (i, 0))
  ospec = pl.BlockSpec((_ROW_BLK, d_out), lambda i: (i, 0))
  return pl.pallas_call(
      _final_body,
      grid=(_NRB,),
      in_specs=[pspec, sspec],
      out_specs=ospec,
      out_shape=jax.ShapeDtypeStruct((n, d_out), jnp.float32),
  )(part, s2)


def _acc_zero(cb, acc, s, width):
  zeros16 = jnp.zeros((16,), jnp.float32)

  def zero_row(r, carry):
    for j in range(width // 16):
      cb[r, pl.ds(j * 16, 16)] = zeros16
    return carry

  lax.fori_loop(0, RCHUNK, zero_row, 0)
  for t in range(CHUNK_ITERS):
    m = s + NS * t
    r0 = pl.multiple_of(m * RCHUNK, RCHUNK)

    @pl.when(m < NCHUNK)
    def _zero_chunk():
      pltpu.sync_copy(cb, acc.at[pl.ds(r0, RCHUNK)])


def _acc_writeout(cb, acc, part_hbm, c, s):
  for t in range(CHUNK_ITERS):
    m = s + NS * t
    r0 = pl.multiple_of(m * RCHUNK, RCHUNK)

    @pl.when(m < NCHUNK)
    def _copy_chunk():
      pltpu.sync_copy(acc.at[pl.ds(r0, RCHUNK)], cb)
      pltpu.sync_copy(cb, part_hbm.at[c, pl.ds(r0, RCHUNK)])


def _make_logits_kernel(d):
  nvec = d // 16
  inv_sqrt_d = float(1.0 / (d ** 0.5))
  mesh = plsc.VectorSubcoreMesh(core_axis_name="c", subcore_axis_name="s",
                                num_cores=NC)

  @functools.partial(
      pl.kernel,
      out_type=[
          jax.ShapeDtypeStruct((NW, NBLK, K), jnp.float32),
          jax.ShapeDtypeStruct((NC, N_NODES, 16), jnp.float32),
      ],
      mesh=mesh,
      compiler_params=_SC_PARAMS,
      scratch_types=[
          pltpu.VMEM((NBLK, K), jnp.int32),
          pltpu.VMEM((NBLK, K), jnp.int32),
          pltpu.VMEM((2, K, d), jnp.float32),
          pltpu.VMEM((2, K, d), jnp.float32),
          pltpu.VMEM((NBLK, K), jnp.float32),
          pltpu.VMEM((K, 16), jnp.float32),
          pltpu.VMEM((RCHUNK, 16), jnp.float32),
          pltpu.VMEM_SHARED((N_NODES, 16), jnp.float32),
          pltpu.SemaphoreType.DMA,
      ],
  )
  def logits_kernel(q_hbm, k_hbm, src_hbm, dst_hbm, ex_hbm, part_hbm,
                    src_i, dst_i, qb, kb, exb, obd, cb, acc, sem):
    c = lax.axis_index("c")
    s = lax.axis_index("s")
    wid = s * NC + c

    pltpu.sync_copy(src_hbm.at[wid], src_i)
    pltpu.sync_copy(dst_hbm.at[wid], dst_i)
    _acc_zero(cb, acc, s, 16)
    plsc.subcore_barrier()

    lane_iota = lax.iota(jnp.int32, 16)
    zeros_i = jnp.zeros((16,), jnp.int32)

    def fire(jj, p):
      pltpu.async_copy(q_hbm.at[dst_i.at[jj]], qb.at[p], sem)
      pltpu.async_copy(k_hbm.at[src_i.at[jj]], kb.at[p], sem)

    def drain():
      pltpu.make_async_copy(q_hbm.at[dst_i.at[0]], qb.at[0], sem).wait()
      pltpu.make_async_copy(k_hbm.at[src_i.at[0]], kb.at[0], sem).wait()

    fire(0, 0)

    def block_body(j, carry):
      p = lax.rem(j, 2)
      dstj = dst_i.at[j]

      @pl.when(j + 1 < NBLK)
      def _fire_next():
        fire(j + 1, 1 - p)

      drain()
      for g in range(KG):
        for l in range(16):
          e = g * 16 + l
          parts = [qb[p, e, pl.ds(dd * 16, 16)] * kb[p, e, pl.ds(dd * 16, 16)]
                   for dd in range(nvec)]
          while len(parts) > 1:
            parts = [parts[i] + parts[i + 1] for i in range(0, len(parts) - 1, 2)] \
                + ([parts[-1]] if len(parts) % 2 else [])
          logit = jnp.sum(parts[0]) * inv_sqrt_d
          obd[e, pl.ds(0, 16)] = jnp.exp(jnp.full((16,), logit, jnp.float32))
        packed = plsc.load_gather(obd, [g * 16 + lane_iota, zeros_i])
        exb[j, pl.ds(g * 16, 16)] = packed
      pltpu.sync_copy(obd, acc.at[dstj], add=True)
      return carry

    lax.fori_loop(0, NBLK, block_body, 0)
    pltpu.sync_copy(exb, ex_hbm.at[wid])
    plsc.subcore_barrier()
    _acc_writeout(cb, acc, part_hbm, c, s)

  return logits_kernel


def _make_scale_scatter_kernel(d):
  h = d // 2
  nvec = h // 16
  nblk2 = N_EDGES // (NS * K)
  mesh = plsc.VectorSubcoreMesh(core_axis_name="c", subcore_axis_name="s",
                                num_cores=NC)

  @functools.partial(
      pl.kernel,
      out_type=jax.ShapeDtypeStruct((NC, N_NODES, h), jnp.float32),
      mesh=mesh,
      compiler_params=_SC_PARAMS,
      scratch_types=[
          pltpu.VMEM((nblk2, K), jnp.int32),
          pltpu.VMEM((nblk2, K), jnp.int32),
          pltpu.VMEM((nblk2, K), jnp.float32),
          pltpu.VMEM((2, K, h), jnp.float32),
          pltpu.VMEM((K, h), jnp.float32),
          pltpu.VMEM((RCHUNK, h), jnp.float32),
          pltpu.VMEM_SHARED((N_NODES, h), jnp.float32),
          pltpu.SemaphoreType.DMA,
      ],
  )
  def scale_scatter_kernel(vlo_hbm, vhi_hbm, src_hbm, dst_hbm, ex_hbm,
                           part_hbm, src_i, dst_i, exv, vb, ob, cb, acc, sem):
    c = lax.axis_index("c")
    s = lax.axis_index("s")

    pltpu.sync_copy(src_hbm.at[s], src_i)
    pltpu.sync_copy(dst_hbm.at[s], dst_i)
    pltpu.sync_copy(ex_hbm.at[s], exv)
    _acc_zero(cb, acc, s, h)
    plsc.subcore_barrier()

    def fire(jj, p):
      srcj = src_i.at[jj]

      @pl.when(c == 0)
      def _gather_lo():
        pltpu.async_copy(vlo_hbm.at[srcj], vb.at[p], sem)

      @pl.when(c != 0)
      def _gather_hi():
        pltpu.async_copy(vhi_hbm.at[srcj], vb.at[p], sem)

    def drain():
      pltpu.make_async_copy(vlo_hbm.at[src_i.at[0]], vb.at[0], sem).wait()

    fire(0, 0)

    def block_body(j, carry):
      p = lax.rem(j, 2)
      dstj = dst_i.at[j]

      @pl.when(j + 1 < nblk2)
      def _fire_next():
        fire(j + 1, 1 - p)

      drain()
      for g in range(KG):
        exvec = exv[j, pl.ds(g * 16, 16)]
        for l in range(16):
          e = g * 16 + l
          exl = jnp.full((16,), exvec[l], jnp.float32)
          for dd in range(nvec):
            ob[e, pl.ds(dd * 16, 16)] = vb[p, e, pl.ds(dd * 16, 16)] * exl
      pltpu.sync_copy(ob, acc.at[dstj], add=True)
      return carry

    lax.fori_loop(0, nblk2, block_body, 0)
    plsc.subcore_barrier()
    _acc_writeout(cb, acc, part_hbm, c, s)

  return scale_scatter_kernel


def _make_fused_edge_kernel(d):
  w = d + 16
  nvec = d // 16
  inv_sqrt_d = float(1.0 / (d ** 0.5))
  mesh = plsc.VectorSubcoreMesh(core_axis_name="c", subcore_axis_name="s",
                                num_cores=NC)

  @functools.partial(
      pl.kernel,
      out_type=jax.ShapeDtypeStruct((NC, N_NODES, w), jnp.float32),
      mesh=mesh,
      compiler_params=_SC_PARAMS,
      scratch_types=[
          pltpu.VMEM((NBLK, K), jnp.int32),
          pltpu.VMEM((NBLK, K), jnp.int32),
          pltpu.VMEM((2, K, d), jnp.float32),
          pltpu.VMEM((2, K, d), jnp.float32),
          pltpu.VMEM((2, K, d), jnp.float32),
          pltpu.VMEM((K, w), jnp.float32),
          pltpu.VMEM((RCHUNK, w), jnp.float32),
          pltpu.VMEM_SHARED((N_NODES, w), jnp.float32),
          pltpu.SemaphoreType.DMA,
      ],
  )
  def fused_kernel(q_hbm, k_hbm, v_hbm, src_hbm, dst_hbm, part_hbm,
                   src_i, dst_i, qb, kb, vb, ob, cb, acc, sem):
    c = lax.axis_index("c")
    s = lax.axis_index("s")
    wid = s * NC + c

    pltpu.sync_copy(src_hbm.at[wid], src_i)
    pltpu.sync_copy(dst_hbm.at[wid], dst_i)
    _acc_zero(cb, acc, s, w)
    plsc.subcore_barrier()

    def fire(jj, p):
      pltpu.async_copy(q_hbm.at[dst_i.at[jj]], qb.at[p], sem)
      pltpu.async_copy(k_hbm.at[src_i.at[jj]], kb.at[p], sem)
      pltpu.async_copy(v_hbm.at[src_i.at[jj]], vb.at[p], sem)

    def drain():
      pltpu.make_async_copy(q_hbm.at[dst_i.at[0]], qb.at[0], sem).wait()
      pltpu.make_async_copy(k_hbm.at[src_i.at[0]], kb.at[0], sem).wait()
      pltpu.make_async_copy(v_hbm.at[src_i.at[0]], vb.at[0], sem).wait()

    fire(0, 0)

    def block_body(j, carry):
      p = lax.rem(j, 2)
      dstj = dst_i.at[j]

      @pl.when(j + 1 < NBLK)
      def _fire_next():
        fire(j + 1, 1 - p)

      drain()
      for e in range(K):
        acc16 = qb[p, e, pl.ds(0, 16)] * kb[p, e, pl.ds(0, 16)]
        for dd in range(1, nvec):
          acc16 = acc16 + qb[p, e, pl.ds(dd * 16, 16)] * kb[p, e, pl.ds(dd * 16, 16)]
        logit = jnp.sum(acc16) * inv_sqrt_d
        ex = jnp.exp(jnp.full((16,), logit, jnp.float32))
        ob[e, pl.ds(0, 16)] = ex
        for dd in range(nvec):
          ob[e, pl.ds(16 + dd * 16, 16)] = vb[p, e, pl.ds(dd * 16, 16)] * ex
      pltpu.sync_copy(ob, acc.at[dstj], add=True)
      return carry

    lax.fori_loop(0, NBLK, block_body, 0)
    plsc.subcore_barrier()
    _acc_writeout(cb, acc, part_hbm, c, s)

  return fused_kernel


_logits_128 = _make_logits_kernel(D_H1)
_scale_scatter_128 = _make_scale_scatter_kernel(D_H1)
_fused_16 = _make_fused_edge_kernel(D_H2)


def kernel(x, train_pos_edge_index,
           Wq1, bq1, Wk1, bk1, Wv1, bv1, Ws1, bs1,
           Wq2, bq2, Wk2, bk2, Wv2, bv2, Ws2, bs2):
  src_flat = train_pos_edge_index[0].astype(jnp.int32)
  dst_flat = train_pos_edge_index[1].astype(jnp.int32)
  src = src_flat.reshape(NW, NBLK, K)
  dst = dst_flat.reshape(NW, NBLK, K)
  nblk2 = N_EDGES // (NS * K)
  src_b = src_flat.reshape(NS, nblk2, K)
  dst_b = dst_flat.reshape(NS, nblk2, K)

  q1, k1, v1lo, v1hi, s1 = _projections(x, Wq1, bq1, Wk1, bk1, Wv1, bv1,
                                        Ws1, bs1)
  ex1, pd1 = _logits_128(q1, k1, src, dst)
  pn1 = _scale_scatter_128(v1lo, v1hi, src_b, dst_b,
                           ex1.reshape(NS, nblk2, K))
  q2, k2, v2, s2 = _combine_projections(pn1, pd1, s1, Wq2, bq2, Wk2, bk2,
                                        Wv2, bv2, Ws2, bs2)
  part2 = _fused_16(q2, k2, v2, src, dst)
  return _final_combine(part2, s2)


# speedup vs baseline: 1.5978x; 1.5978x over previous
import functools

import jax
import jax.numpy as jnp
from jax import lax
from jax.experimental import pallas as pl
from jax.experimental.pallas import tpu as pltpu
from jax.experimental.pallas import tpu_sc as plsc

N_NODES = 10000
N_EDGES = 320000
D_IN = 128
D_H1 = 128
D_H2 = 16

NC, NS = 2, 16
NW = NC * NS
EPT = N_EDGES // NW
K = 80
KG = K // 16
NBLK = EPT // K
RCHUNK = 200
NCHUNK = N_NODES // RCHUNK
CHUNK_ITERS = -(-NCHUNK // NS)

_ROW_BLK = 1000
_NRB = N_NODES // _ROW_BLK

_SC_PARAMS = pltpu.CompilerParams(
    needs_layout_passes=False, use_tc_tiling_on_sc=False)


def _proj_body(x_ref, wq, bq, wk, bk, wv, bv, ws, bs,
               q_ref, k_ref, vlo_ref, vhi_ref, s_ref):
  xb = x_ref[...]
  q_ref[...] = jnp.dot(xb, wq[...], preferred_element_type=jnp.float32) + bq[...]
  k_ref[...] = jnp.dot(xb, wk[...], preferred_element_type=jnp.float32) + bk[...]
  v = jnp.dot(xb, wv[...], preferred_element_type=jnp.float32) + bv[...]
  half = v.shape[1] // 2
  vlo_ref[...] = v[:, :half]
  vhi_ref[...] = v[:, half:]
  s_ref[...] = jnp.dot(xb, ws[...], preferred_element_type=jnp.float32) + bs[...]


def _projections(x, wq, bq, wk, bk, wv, bv, ws, bs):
  n, d_in = x.shape
  d_out = wq.shape[1]
  half = d_out // 2
  row = pl.BlockSpec((_ROW_BLK, d_in), lambda i: (i, 0))
  wspec = pl.BlockSpec((d_in, d_out), lambda i: (0, 0))
  bspec = pl.BlockSpec((1, d_out), lambda i: (0, 0))
  ospec = pl.BlockSpec((_ROW_BLK, d_out), lambda i: (i, 0))
  hspec = pl.BlockSpec((_ROW_BLK, half), lambda i: (i, 0))
  out = jax.ShapeDtypeStruct((n, d_out), jnp.float32)
  outh = jax.ShapeDtypeStruct((n, half), jnp.float32)
  return pl.pallas_call(
      _proj_body,
      grid=(_NRB,),
      in_specs=[row, wspec, bspec, wspec, bspec, wspec, bspec, wspec, bspec],
      out_specs=[ospec, ospec, hspec, hspec, ospec],
      out_shape=[out, out, outh, outh, out],
  )(x, wq, bq.reshape(1, -1), wk, bk.reshape(1, -1),
    wv, bv.reshape(1, -1), ws, bs.reshape(1, -1))


def _combine_proj_body(pn_ref, pd_ref, s1_ref, wq, bq, wk, bk, wv, bv, ws, bs,
                       q_ref, k_ref, v_ref, s_ref):
  numer = jnp.concatenate([pn_ref[i] for i in range(NC)], axis=1)
  denom = pd_ref[0, :, 0:1]
  for i in range(1, NC):
    denom = denom + pd_ref[i, :, 0:1]
  h = jnp.maximum(numer / (denom + 1e-16) + s1_ref[...], 0.0)
  q_ref[...] = jnp.dot(h, wq[...], preferred_element_type=jnp.float32) + bq[...]
  k_ref[...] = jnp.dot(h, wk[...], preferred_element_type=jnp.float32) + bk[...]
  v_ref[...] = jnp.dot(h, wv[...], preferred_element_type=jnp.float32) + bv[...]
  s_ref[...] = jnp.dot(h, ws[...], preferred_element_type=jnp.float32) + bs[...]


def _combine_projections(pn, pd, s1, wq, bq, wk, bk, wv, bv, ws, bs):
  n, d_in = s1.shape
  d_out = wq.shape[1]
  pnspec = pl.BlockSpec((NC, _ROW_BLK, d_in // NC), lambda i: (0, i, 0))
  pdspec = pl.BlockSpec((NC, _ROW_BLK, 16), lambda i: (0, i, 0))
  sspec = pl.BlockSpec((_ROW_BLK, d_in), lambda i: (i, 0))
  wspec = pl.BlockSpec((d_in, d_out), lambda i: (0, 0))
  bspec = pl.BlockSpec((1, d_out), lambda i: (0, 0))
  ospec = pl.BlockSpec((_ROW_BLK, d_out), lambda i: (i, 0))
  out = jax.ShapeDtypeStruct((n, d_out), jnp.float32)
  return pl.pallas_call(
      _combine_proj_body,
      grid=(_NRB,),
      in_specs=[pnspec, pdspec, sspec, wspec, bspec, wspec, bspec,
                wspec, bspec, wspec, bspec],
      out_specs=[ospec, ospec, ospec, ospec],
      out_shape=[out, out, out, out],
  )(pn, pd, s1, wq, bq.reshape(1, -1), wk, bk.reshape(1, -1),
    wv, bv.reshape(1, -1), ws, bs.reshape(1, -1))


def _final_body(part_ref, s2_ref, out_ref):
  p = part_ref[0]
  for i in range(1, NC):
    p = p + part_ref[i]
  denom = p[:, 0:1]
  numer = p[:, 16:]
  out_ref[...] = numer / (denom + 1e-16) + s2_ref[...]


def _final_combine(part, s2):
  n, d_out = s2.shape
  w_in = part.shape[2]
  pspec = pl.BlockSpec((NC, _ROW_BLK, w_in), lambda i: (0, i, 0))
  sspec = pl.BlockSpec((_ROW_BLK, d_out), lambda i: (i, 0))
  ospec = pl.BlockSpec((_ROW_BLK, d_out), lambda i: (i, 0))
  return pl.pallas_call(
      _final_body,
      grid=(_NRB,),
      in_specs=[pspec, sspec],
      out_specs=ospec,
      out_shape=jax.ShapeDtypeStruct((n, d_out), jnp.float32),
  )(part, s2)


def _acc_zero(cb, acc, s, width):
  zeros16 = jnp.zeros((16,), jnp.float32)

  def zero_row(r, carry):
    for j in range(width // 16):
      cb[r, pl.ds(j * 16, 16)] = zeros16
    return carry

  lax.fori_loop(0, RCHUNK, zero_row, 0)
  for t in range(CHUNK_ITERS):
    m = s + NS * t
    r0 = pl.multiple_of(m * RCHUNK, RCHUNK)

    @pl.when(m < NCHUNK)
    def _zero_chunk():
      pltpu.sync_copy(cb, acc.at[pl.ds(r0, RCHUNK)])


def _acc_writeout(cb, acc, part_hbm, c, s):
  for t in range(CHUNK_ITERS):
    m = s + NS * t
    r0 = pl.multiple_of(m * RCHUNK, RCHUNK)

    @pl.when(m < NCHUNK)
    def _copy_chunk():
      pltpu.sync_copy(acc.at[pl.ds(r0, RCHUNK)], cb)
      pltpu.sync_copy(cb, part_hbm.at[c, pl.ds(r0, RCHUNK)])


def _make_logits_kernel(d):
  nvec = d // 16
  inv_sqrt_d = float(1.0 / (d ** 0.5))
  mesh = plsc.VectorSubcoreMesh(core_axis_name="c", subcore_axis_name="s",
                                num_cores=NC)

  @functools.partial(
      pl.kernel,
      out_type=[
          jax.ShapeDtypeStruct((NW, NBLK, K), jnp.float32),
          jax.ShapeDtypeStruct((NC, N_NODES, 16), jnp.float32),
      ],
      mesh=mesh,
      compiler_params=_SC_PARAMS,
      scratch_types=[
          pltpu.VMEM((NBLK, K), jnp.int32),
          pltpu.VMEM((NBLK, K), jnp.int32),
          pltpu.VMEM((K, d), jnp.float32),
          pltpu.VMEM((K, d), jnp.float32),
          pltpu.VMEM((NBLK, K), jnp.float32),
          pltpu.VMEM((K, 16), jnp.float32),
          pltpu.VMEM((RCHUNK, 16), jnp.float32),
          pltpu.VMEM_SHARED((N_NODES, 16), jnp.float32),
          pltpu.SemaphoreType.DMA,
      ],
  )
  def logits_kernel(q_hbm, k_hbm, src_hbm, dst_hbm, ex_hbm, part_hbm,
                    src_i, dst_i, qb, kb, exb, obd, cb, acc, sem):
    c = lax.axis_index("c")
    s = lax.axis_index("s")
    wid = s * NC + c

    pltpu.sync_copy(src_hbm.at[wid], src_i)
    pltpu.sync_copy(dst_hbm.at[wid], dst_i)
    _acc_zero(cb, acc, s, 16)
    plsc.subcore_barrier()

    lane_iota = lax.iota(jnp.int32, 16)
    zeros_i = jnp.zeros((16,), jnp.int32)

    def block_body(j, carry):
      dstj = dst_i.at[j]
      dq = pltpu.async_copy(q_hbm.at[dstj], qb, sem)
      dk = pltpu.async_copy(k_hbm.at[src_i.at[j]], kb, sem)
      dq.wait()
      dk.wait()
      for g in range(KG):
        for l in range(16):
          e = g * 16 + l
          parts = [qb[e, pl.ds(dd * 16, 16)] * kb[e, pl.ds(dd * 16, 16)]
                   for dd in range(nvec)]
          while len(parts) > 1:
            parts = [parts[i] + parts[i + 1] for i in range(0, len(parts) - 1, 2)] \
                + ([parts[-1]] if len(parts) % 2 else [])
          logit = jnp.sum(parts[0]) * inv_sqrt_d
          obd[e, pl.ds(0, 16)] = jnp.exp(jnp.full((16,), logit, jnp.float32))
        packed = plsc.load_gather(obd, [g * 16 + lane_iota, zeros_i])
        exb[j, pl.ds(g * 16, 16)] = packed
      pltpu.sync_copy(obd, acc.at[dstj], add=True)
      return carry

    lax.fori_loop(0, NBLK, block_body, 0)
    pltpu.sync_copy(exb, ex_hbm.at[wid])
    plsc.subcore_barrier()
    _acc_writeout(cb, acc, part_hbm, c, s)

  return logits_kernel


def _make_scale_scatter_kernel(d):
  h = d // 2
  nvec = h // 16
  nblk2 = N_EDGES // (NS * K)
  mesh = plsc.VectorSubcoreMesh(core_axis_name="c", subcore_axis_name="s",
                                num_cores=NC)

  @functools.partial(
      pl.kernel,
      out_type=jax.ShapeDtypeStruct((NC, N_NODES, h), jnp.float32),
      mesh=mesh,
      compiler_params=_SC_PARAMS,
      scratch_types=[
          pltpu.VMEM((nblk2, K), jnp.int32),
          pltpu.VMEM((nblk2, K), jnp.int32),
          pltpu.VMEM((nblk2, K), jnp.float32),
          pltpu.VMEM((K, h), jnp.float32),
          pltpu.VMEM((K, h), jnp.float32),
          pltpu.VMEM((K, h), jnp.float32),
          pltpu.VMEM((RCHUNK, h), jnp.float32),
          pltpu.VMEM_SHARED((N_NODES, h), jnp.float32),
          pltpu.SemaphoreType.DMA,
      ],
  )
  def scale_scatter_kernel(vlo_hbm, vhi_hbm, src_hbm, dst_hbm, ex_hbm,
                           part_hbm, src_i, dst_i, exv, vb_a, vb_b, ob,
                           cb, acc, sem):
    c = lax.axis_index("c")
    s = lax.axis_index("s")

    pltpu.sync_copy(src_hbm.at[s], src_i)
    pltpu.sync_copy(dst_hbm.at[s], dst_i)
    pltpu.sync_copy(ex_hbm.at[s], exv)
    _acc_zero(cb, acc, s, h)
    plsc.subcore_barrier()

    def fire(jj, vb_):
      srcj = src_i.at[jj]

      @pl.when(c == 0)
      def _gather_lo():
        pltpu.async_copy(vlo_hbm.at[srcj], vb_, sem)

      @pl.when(c != 0)
      def _gather_hi():
        pltpu.async_copy(vhi_hbm.at[srcj], vb_, sem)

    def drain():
      pltpu.make_async_copy(vlo_hbm.at[src_i.at[0]], vb_a, sem).wait()

    def compute_block(j, vb_):
      dstj = dst_i.at[j]
      for g in range(KG):
        exvec = exv[j, pl.ds(g * 16, 16)]
        for l in range(16):
          e = g * 16 + l
          exl = jnp.full((16,), exvec[l], jnp.float32)
          for dd in range(nvec):
            ob[e, pl.ds(dd * 16, 16)] = vb_[e, pl.ds(dd * 16, 16)] * exl
      pltpu.sync_copy(ob, acc.at[dstj], add=True)

    fire(0, vb_a)

    def pair_body(m, carry):
      j0 = m * 2
      fire(j0 + 1, vb_b)
      drain()
      compute_block(j0, vb_a)

      @pl.when(j0 + 2 < nblk2)
      def _fire_next():
        fire(j0 + 2, vb_a)

      drain()
      compute_block(j0 + 1, vb_b)
      return carry

    lax.fori_loop(0, nblk2 // 2, pair_body, 0)
    plsc.subcore_barrier()
    _acc_writeout(cb, acc, part_hbm, c, s)

  return scale_scatter_kernel


def _make_fused_edge_kernel(d):
  w = d + 16
  nvec = d // 16
  inv_sqrt_d = float(1.0 / (d ** 0.5))
  mesh = plsc.VectorSubcoreMesh(core_axis_name="c", subcore_axis_name="s",
                                num_cores=NC)

  @functools.partial(
      pl.kernel,
      out_type=jax.ShapeDtypeStruct((NC, N_NODES, w), jnp.float32),
      mesh=mesh,
      compiler_params=_SC_PARAMS,
      scratch_types=[
          pltpu.VMEM((NBLK, K), jnp.int32),
          pltpu.VMEM((NBLK, K), jnp.int32),
          pltpu.VMEM((2, K, d), jnp.float32),
          pltpu.VMEM((2, K, d), jnp.float32),
          pltpu.VMEM((2, K, d), jnp.float32),
          pltpu.VMEM((K, w), jnp.float32),
          pltpu.VMEM((RCHUNK, w), jnp.float32),
          pltpu.VMEM_SHARED((N_NODES, w), jnp.float32),
          pltpu.SemaphoreType.DMA,
      ],
  )
  def fused_kernel(q_hbm, k_hbm, v_hbm, src_hbm, dst_hbm, part_hbm,
                   src_i, dst_i, qb, kb, vb, ob, cb, acc, sem):
    c = lax.axis_index("c")
    s = lax.axis_index("s")
    wid = s * NC + c

    pltpu.sync_copy(src_hbm.at[wid], src_i)
    pltpu.sync_copy(dst_hbm.at[wid], dst_i)
    _acc_zero(cb, acc, s, w)
    plsc.subcore_barrier()

    def fire(jj, p):
      pltpu.async_copy(q_hbm.at[dst_i.at[jj]], qb.at[p], sem)
      pltpu.async_copy(k_hbm.at[src_i.at[jj]], kb.at[p], sem)
      pltpu.async_copy(v_hbm.at[src_i.at[jj]], vb.at[p], sem)

    def drain():
      pltpu.make_async_copy(q_hbm.at[dst_i.at[0]], qb.at[0], sem).wait()
      pltpu.make_async_copy(k_hbm.at[src_i.at[0]], kb.at[0], sem).wait()
      pltpu.make_async_copy(v_hbm.at[src_i.at[0]], vb.at[0], sem).wait()

    def compute_block(j, p):
      dstj = dst_i.at[j]
      for e in range(K):
        acc16 = qb[p, e, pl.ds(0, 16)] * kb[p, e, pl.ds(0, 16)]
        for dd in range(1, nvec):
          acc16 = acc16 + qb[p, e, pl.ds(dd * 16, 16)] * kb[p, e, pl.ds(dd * 16, 16)]
        logit = jnp.sum(acc16) * inv_sqrt_d
        ex = jnp.exp(jnp.full((16,), logit, jnp.float32))
        ob[e, pl.ds(0, 16)] = ex
        for dd in range(nvec):
          ob[e, pl.ds(16 + dd * 16, 16)] = vb[p, e, pl.ds(dd * 16, 16)] * ex
      pltpu.sync_copy(ob, acc.at[dstj], add=True)

    fire(0, 0)

    def pair_body(m, carry):
      j0 = m * 2
      fire(j0 + 1, 1)
      drain()
      compute_block(j0, 0)

      @pl.when(j0 + 2 < NBLK)
      def _fire_next():
        fire(j0 + 2, 0)

      drain()
      compute_block(j0 + 1, 1)
      return carry

    lax.fori_loop(0, NBLK // 2, pair_body, 0)
    drain()
    compute_block(NBLK - 1, 0)
    plsc.subcore_barrier()
    _acc_writeout(cb, acc, part_hbm, c, s)

  return fused_kernel


_logits_128 = _make_logits_kernel(D_H1)
_scale_scatter_128 = _make_scale_scatter_kernel(D_H1)
_fused_16 = _make_fused_edge_kernel(D_H2)


def kernel(x, train_pos_edge_index,
           Wq1, bq1, Wk1, bk1, Wv1, bv1, Ws1, bs1,
           Wq2, bq2, Wk2, bk2, Wv2, bv2, Ws2, bs2):
  src_flat = train_pos_edge_index[0].astype(jnp.int32)
  dst_flat = train_pos_edge_index[1].astype(jnp.int32)
  src = src_flat.reshape(NW, NBLK, K)
  dst = dst_flat.reshape(NW, NBLK, K)
  nblk2 = N_EDGES // (NS * K)
  src_b = src_flat.reshape(NS, nblk2, K)
  dst_b = dst_flat.reshape(NS, nblk2, K)

  q1, k1, v1lo, v1hi, s1 = _projections(x, Wq1, bq1, Wk1, bk1, Wv1, bv1,
                                        Ws1, bs1)
  ex1, pd1 = _logits_128(q1, k1, src, dst)
  pn1 = _scale_scatter_128(v1lo, v1hi, src_b, dst_b,
                           ex1.reshape(NS, nblk2, K))
  q2, k2, v2, s2 = _combine_projections(pn1, pd1, s1, Wq2, bq2, Wk2, bk2,
                                        Wv2, bv2, Ws2, bs2)
  part2 = _fused_16(q2, k2, v2, src, dst)
  return _final_combine(part2, s2)


# speedup vs baseline: 1.8998x; 1.1890x over previous
import functools

import jax
import jax.numpy as jnp
from jax import lax
from jax.experimental import pallas as pl
from jax.experimental.pallas import tpu as pltpu
from jax.experimental.pallas import tpu_sc as plsc

N_NODES = 10000
N_EDGES = 320000
D_IN = 128
D_H1 = 128
D_H2 = 16

NC, NS = 2, 16
NW = NC * NS
EPT = N_EDGES // NW
K = 80
KG = K // 16
NBLK = EPT // K
RCHUNK = 200
NCHUNK = N_NODES // RCHUNK
CHUNK_ITERS = -(-NCHUNK // NS)

_ROW_BLK = 1000
_NRB = N_NODES // _ROW_BLK

_SC_PARAMS = pltpu.CompilerParams(
    needs_layout_passes=False, use_tc_tiling_on_sc=False)


def _proj_body(x_ref, wq, bq, wk, bk, wv, bv, ws, bs,
               q_ref, k_ref, vlo_ref, vhi_ref, s_ref):
  xb = x_ref[...]
  q_ref[...] = jnp.dot(xb, wq[...], preferred_element_type=jnp.float32) + bq[...]
  k_ref[...] = jnp.dot(xb, wk[...], preferred_element_type=jnp.float32) + bk[...]
  v = jnp.dot(xb, wv[...], preferred_element_type=jnp.float32) + bv[...]
  half = v.shape[1] // 2
  vlo_ref[...] = v[:, :half]
  vhi_ref[...] = v[:, half:]
  s_ref[...] = jnp.dot(xb, ws[...], preferred_element_type=jnp.float32) + bs[...]


def _projections(x, wq, bq, wk, bk, wv, bv, ws, bs):
  n, d_in = x.shape
  d_out = wq.shape[1]
  half = d_out // 2
  row = pl.BlockSpec((_ROW_BLK, d_in), lambda i: (i, 0))
  wspec = pl.BlockSpec((d_in, d_out), lambda i: (0, 0))
  bspec = pl.BlockSpec((1, d_out), lambda i: (0, 0))
  ospec = pl.BlockSpec((_ROW_BLK, d_out), lambda i: (i, 0))
  hspec = pl.BlockSpec((_ROW_BLK, half), lambda i: (i, 0))
  out = jax.ShapeDtypeStruct((n, d_out), jnp.float32)
  outh = jax.ShapeDtypeStruct((n, half), jnp.float32)
  return pl.pallas_call(
      _proj_body,
      grid=(_NRB,),
      in_specs=[row, wspec, bspec, wspec, bspec, wspec, bspec, wspec, bspec],
      out_specs=[ospec, ospec, hspec, hspec, ospec],
      out_shape=[out, out, outh, outh, out],
  )(x, wq, bq.reshape(1, -1), wk, bk.reshape(1, -1),
    wv, bv.reshape(1, -1), ws, bs.reshape(1, -1))


def _combine_proj_body(pn_ref, pd_ref, s1_ref, wq, bq, wk, bk, wv, bv, ws, bs,
                       q_ref, k_ref, v_ref, s_ref):
  numer = jnp.concatenate([pn_ref[i] for i in range(NC)], axis=1)
  denom = pd_ref[0, :, 0:1]
  for i in range(1, NC):
    denom = denom + pd_ref[i, :, 0:1]
  h = jnp.maximum(numer / (denom + 1e-16) + s1_ref[...], 0.0)
  q_ref[...] = jnp.dot(h, wq[...], preferred_element_type=jnp.float32) + bq[...]
  k_ref[...] = jnp.dot(h, wk[...], preferred_element_type=jnp.float32) + bk[...]
  v_ref[...] = jnp.dot(h, wv[...], preferred_element_type=jnp.float32) + bv[...]
  s_ref[...] = jnp.dot(h, ws[...], preferred_element_type=jnp.float32) + bs[...]


def _combine_projections(pn, pd, s1, wq, bq, wk, bk, wv, bv, ws, bs):
  n, d_in = s1.shape
  d_out = wq.shape[1]
  pnspec = pl.BlockSpec((NC, _ROW_BLK, d_in // NC), lambda i: (0, i, 0))
  pdspec = pl.BlockSpec((NC, _ROW_BLK, 16), lambda i: (0, i, 0))
  sspec = pl.BlockSpec((_ROW_BLK, d_in), lambda i: (i, 0))
  wspec = pl.BlockSpec((d_in, d_out), lambda i: (0, 0))
  bspec = pl.BlockSpec((1, d_out), lambda i: (0, 0))
  ospec = pl.BlockSpec((_ROW_BLK, d_out), lambda i: (i, 0))
  out = jax.ShapeDtypeStruct((n, d_out), jnp.float32)
  return pl.pallas_call(
      _combine_proj_body,
      grid=(_NRB,),
      in_specs=[pnspec, pdspec, sspec, wspec, bspec, wspec, bspec,
                wspec, bspec, wspec, bspec],
      out_specs=[ospec, ospec, ospec, ospec],
      out_shape=[out, out, out, out],
  )(pn, pd, s1, wq, bq.reshape(1, -1), wk, bk.reshape(1, -1),
    wv, bv.reshape(1, -1), ws, bs.reshape(1, -1))


def _final_body(part_ref, s2_ref, out_ref):
  p = part_ref[0]
  for i in range(1, NC):
    p = p + part_ref[i]
  denom = p[:, 0:1]
  numer = p[:, 16:]
  out_ref[...] = numer / (denom + 1e-16) + s2_ref[...]


def _final_combine(part, s2):
  n, d_out = s2.shape
  w_in = part.shape[2]
  pspec = pl.BlockSpec((NC, _ROW_BLK, w_in), lambda i: (0, i, 0))
  sspec = pl.BlockSpec((_ROW_BLK, d_out), lambda i: (i, 0))
  ospec = pl.BlockSpec((_ROW_BLK, d_out), lambda i: (i, 0))
  return pl.pallas_call(
      _final_body,
      grid=(_NRB,),
      in_specs=[pspec, sspec],
      out_specs=ospec,
      out_shape=jax.ShapeDtypeStruct((n, d_out), jnp.float32),
  )(part, s2)


def _acc_zero(cb, acc, s, width):
  zeros16 = jnp.zeros((16,), jnp.float32)

  def zero_row(r, carry):
    for j in range(width // 16):
      cb[r, pl.ds(j * 16, 16)] = zeros16
    return carry

  lax.fori_loop(0, RCHUNK, zero_row, 0)
  for t in range(CHUNK_ITERS):
    m = s + NS * t
    r0 = pl.multiple_of(m * RCHUNK, RCHUNK)

    @pl.when(m < NCHUNK)
    def _zero_chunk():
      pltpu.sync_copy(cb, acc.at[pl.ds(r0, RCHUNK)])


def _acc_writeout(cb, acc, part_hbm, c, s):
  for t in range(CHUNK_ITERS):
    m = s + NS * t
    r0 = pl.multiple_of(m * RCHUNK, RCHUNK)

    @pl.when(m < NCHUNK)
    def _copy_chunk():
      pltpu.sync_copy(acc.at[pl.ds(r0, RCHUNK)], cb)
      pltpu.sync_copy(cb, part_hbm.at[c, pl.ds(r0, RCHUNK)])


def _make_logits_kernel(d):
  nvec = d // 16
  inv_sqrt_d = float(1.0 / (d ** 0.5))
  mesh = plsc.VectorSubcoreMesh(core_axis_name="c", subcore_axis_name="s",
                                num_cores=NC)

  @functools.partial(
      pl.kernel,
      out_type=[
          jax.ShapeDtypeStruct((NW, NBLK, K), jnp.float32),
          jax.ShapeDtypeStruct((NC, N_NODES, 16), jnp.float32),
      ],
      mesh=mesh,
      compiler_params=_SC_PARAMS,
      scratch_types=[
          pltpu.VMEM((NBLK, K), jnp.int32),
          pltpu.VMEM((NBLK, K), jnp.int32),
          pltpu.VMEM((2, K, d), jnp.float32),
          pltpu.VMEM((2, K, d), jnp.float32),
          pltpu.VMEM((NBLK, K), jnp.float32),
          pltpu.VMEM((K, 16), jnp.float32),
          pltpu.VMEM((RCHUNK, 16), jnp.float32),
          pltpu.VMEM_SHARED((N_NODES, 16), jnp.float32),
          pltpu.SemaphoreType.DMA,
      ],
  )
  def logits_kernel(q_hbm, k_hbm, src_hbm, dst_hbm, ex_hbm, part_hbm,
                    src_i, dst_i, qb, kb, exb, obd, cb, acc, sem):
    c = lax.axis_index("c")
    s = lax.axis_index("s")
    wid = s * NC + c

    pltpu.sync_copy(src_hbm.at[wid], src_i)
    pltpu.sync_copy(dst_hbm.at[wid], dst_i)
    _acc_zero(cb, acc, s, 16)
    plsc.subcore_barrier()

    lane_iota = lax.iota(jnp.int32, 16)
    zeros_i = jnp.zeros((16,), jnp.int32)

    def fire(jj, p):
      pltpu.async_copy(q_hbm.at[dst_i.at[jj]], qb.at[p], sem)
      pltpu.async_copy(k_hbm.at[src_i.at[jj]], kb.at[p], sem)

    def drain():
      pltpu.make_async_copy(q_hbm.at[dst_i.at[0]], qb.at[0], sem).wait()
      pltpu.make_async_copy(k_hbm.at[src_i.at[0]], kb.at[0], sem).wait()

    def compute_block(j, p):
      dstj = dst_i.at[j]
      for g in range(KG):
        for l in range(16):
          e = g * 16 + l
          parts = [qb[p, e, pl.ds(dd * 16, 16)] * kb[p, e, pl.ds(dd * 16, 16)]
                   for dd in range(nvec)]
          while len(parts) > 1:
            parts = [parts[i] + parts[i + 1] for i in range(0, len(parts) - 1, 2)] \
                + ([parts[-1]] if len(parts) % 2 else [])
          logit = jnp.sum(parts[0]) * inv_sqrt_d
          obd[e, pl.ds(0, 16)] = jnp.exp(jnp.full((16,), logit, jnp.float32))
        packed = plsc.load_gather(obd, [g * 16 + lane_iota, zeros_i])
        exb[j, pl.ds(g * 16, 16)] = packed
      pltpu.sync_copy(obd, acc.at[dstj], add=True)

    fire(0, 0)

    def pair_body(m, carry):
      j0 = m * 2
      fire(j0 + 1, 1)
      drain()
      compute_block(j0, 0)

      @pl.when(j0 + 2 < NBLK)
      def _fire_next():
        fire(j0 + 2, 0)

      drain()
      compute_block(j0 + 1, 1)
      return carry

    lax.fori_loop(0, NBLK // 2, pair_body, 0)
    drain()
    compute_block(NBLK - 1, 0)
    pltpu.sync_copy(exb, ex_hbm.at[wid])
    plsc.subcore_barrier()
    _acc_writeout(cb, acc, part_hbm, c, s)

  return logits_kernel


def _make_scale_scatter_kernel(d):
  h = d // 2
  nvec = h // 16
  nblk2 = N_EDGES // (NS * K)
  mesh = plsc.VectorSubcoreMesh(core_axis_name="c", subcore_axis_name="s",
                                num_cores=NC)

  @functools.partial(
      pl.kernel,
      out_type=jax.ShapeDtypeStruct((NC, N_NODES, h), jnp.float32),
      mesh=mesh,
      compiler_params=_SC_PARAMS,
      scratch_types=[
          pltpu.VMEM((nblk2, K), jnp.int32),
          pltpu.VMEM((nblk2, K), jnp.int32),
          pltpu.VMEM((nblk2, K), jnp.float32),
          pltpu.VMEM((K, h), jnp.float32),
          pltpu.VMEM((K, h), jnp.float32),
          pltpu.VMEM((K, h), jnp.float32),
          pltpu.VMEM((RCHUNK, h), jnp.float32),
          pltpu.VMEM_SHARED((N_NODES, h), jnp.float32),
          pltpu.SemaphoreType.DMA,
      ],
  )
  def scale_scatter_kernel(vlo_hbm, vhi_hbm, src_hbm, dst_hbm, ex_hbm,
                           part_hbm, src_i, dst_i, exv, vb_a, vb_b, ob,
                           cb, acc, sem):
    c = lax.axis_index("c")
    s = lax.axis_index("s")

    pltpu.sync_copy(src_hbm.at[s], src_i)
    pltpu.sync_copy(dst_hbm.at[s], dst_i)
    pltpu.sync_copy(ex_hbm.at[s], exv)
    _acc_zero(cb, acc, s, h)
    plsc.subcore_barrier()

    def fire(jj, vb_):
      srcj = src_i.at[jj]

      @pl.when(c == 0)
      def _gather_lo():
        pltpu.async_copy(vlo_hbm.at[srcj], vb_, sem)

      @pl.when(c != 0)
      def _gather_hi():
        pltpu.async_copy(vhi_hbm.at[srcj], vb_, sem)

    def drain():
      pltpu.make_async_copy(vlo_hbm.at[src_i.at[0]], vb_a, sem).wait()

    def compute_block(j, vb_):
      dstj = dst_i.at[j]
      for g in range(KG):
        exvec = exv[j, pl.ds(g * 16, 16)]
        for l in range(16):
          e = g * 16 + l
          exl = jnp.full((16,), exvec[l], jnp.float32)
          for dd in range(nvec):
            ob[e, pl.ds(dd * 16, 16)] = vb_[e, pl.ds(dd * 16, 16)] * exl
      pltpu.sync_copy(ob, acc.at[dstj], add=True)

    fire(0, vb_a)

    def pair_body(m, carry):
      j0 = m * 2
      fire(j0 + 1, vb_b)
      drain()
      compute_block(j0, vb_a)

      @pl.when(j0 + 2 < nblk2)
      def _fire_next():
        fire(j0 + 2, vb_a)

      drain()
      compute_block(j0 + 1, vb_b)
      return carry

    lax.fori_loop(0, nblk2 // 2, pair_body, 0)
    plsc.subcore_barrier()
    _acc_writeout(cb, acc, part_hbm, c, s)

  return scale_scatter_kernel


def _make_fused_edge_kernel(d):
  w = d + 16
  nvec = d // 16
  inv_sqrt_d = float(1.0 / (d ** 0.5))
  mesh = plsc.VectorSubcoreMesh(core_axis_name="c", subcore_axis_name="s",
                                num_cores=NC)

  @functools.partial(
      pl.kernel,
      out_type=jax.ShapeDtypeStruct((NC, N_NODES, w), jnp.float32),
      mesh=mesh,
      compiler_params=_SC_PARAMS,
      scratch_types=[
          pltpu.VMEM((NBLK, K), jnp.int32),
          pltpu.VMEM((NBLK, K), jnp.int32),
          pltpu.VMEM((2, K, d), jnp.float32),
          pltpu.VMEM((2, K, d), jnp.float32),
          pltpu.VMEM((2, K, d), jnp.float32),
          pltpu.VMEM((K, w), jnp.float32),
          pltpu.VMEM((RCHUNK, w), jnp.float32),
          pltpu.VMEM_SHARED((N_NODES, w), jnp.float32),
          pltpu.SemaphoreType.DMA,
      ],
  )
  def fused_kernel(q_hbm, k_hbm, v_hbm, src_hbm, dst_hbm, part_hbm,
                   src_i, dst_i, qb, kb, vb, ob, cb, acc, sem):
    c = lax.axis_index("c")
    s = lax.axis_index("s")
    wid = s * NC + c

    pltpu.sync_copy(src_hbm.at[wid], src_i)
    pltpu.sync_copy(dst_hbm.at[wid], dst_i)
    _acc_zero(cb, acc, s, w)
    plsc.subcore_barrier()

    def fire(jj, p):
      pltpu.async_copy(q_hbm.at[dst_i.at[jj]], qb.at[p], sem)
      pltpu.async_copy(k_hbm.at[src_i.at[jj]], kb.at[p], sem)
      pltpu.async_copy(v_hbm.at[src_i.at[jj]], vb.at[p], sem)

    def drain():
      pltpu.make_async_copy(q_hbm.at[dst_i.at[0]], qb.at[0], sem).wait()
      pltpu.make_async_copy(k_hbm.at[src_i.at[0]], kb.at[0], sem).wait()
      pltpu.make_async_copy(v_hbm.at[src_i.at[0]], vb.at[0], sem).wait()

    def compute_block(j, p):
      dstj = dst_i.at[j]
      for e in range(K):
        acc16 = qb[p, e, pl.ds(0, 16)] * kb[p, e, pl.ds(0, 16)]
        for dd in range(1, nvec):
          acc16 = acc16 + qb[p, e, pl.ds(dd * 16, 16)] * kb[p, e, pl.ds(dd * 16, 16)]
        logit = jnp.sum(acc16) * inv_sqrt_d
        ex = jnp.exp(jnp.full((16,), logit, jnp.float32))
        ob[e, pl.ds(0, 16)] = ex
        for dd in range(nvec):
          ob[e, pl.ds(16 + dd * 16, 16)] = vb[p, e, pl.ds(dd * 16, 16)] * ex
      pltpu.sync_copy(ob, acc.at[dstj], add=True)

    fire(0, 0)

    def pair_body(m, carry):
      j0 = m * 2
      fire(j0 + 1, 1)
      drain()
      compute_block(j0, 0)

      @pl.when(j0 + 2 < NBLK)
      def _fire_next():
        fire(j0 + 2, 0)

      drain()
      compute_block(j0 + 1, 1)
      return carry

    lax.fori_loop(0, NBLK // 2, pair_body, 0)
    drain()
    compute_block(NBLK - 1, 0)
    plsc.subcore_barrier()
    _acc_writeout(cb, acc, part_hbm, c, s)

  return fused_kernel


_logits_128 = _make_logits_kernel(D_H1)
_scale_scatter_128 = _make_scale_scatter_kernel(D_H1)
_fused_16 = _make_fused_edge_kernel(D_H2)


def kernel(x, train_pos_edge_index,
           Wq1, bq1, Wk1, bk1, Wv1, bv1, Ws1, bs1,
           Wq2, bq2, Wk2, bk2, Wv2, bv2, Ws2, bs2):
  src_flat = train_pos_edge_index[0].astype(jnp.int32)
  dst_flat = train_pos_edge_index[1].astype(jnp.int32)
  src = src_flat.reshape(NW, NBLK, K)
  dst = dst_flat.reshape(NW, NBLK, K)
  nblk2 = N_EDGES // (NS * K)
  src_b = src_flat.reshape(NS, nblk2, K)
  dst_b = dst_flat.reshape(NS, nblk2, K)

  q1, k1, v1lo, v1hi, s1 = _projections(x, Wq1, bq1, Wk1, bk1, Wv1, bv1,
                                        Ws1, bs1)
  ex1, pd1 = _logits_128(q1, k1, src, dst)
  pn1 = _scale_scatter_128(v1lo, v1hi, src_b, dst_b,
                           ex1.reshape(NS, nblk2, K))
  q2, k2, v2, s2 = _combine_projections(pn1, pd1, s1, Wq2, bq2, Wk2, bk2,
                                        Wv2, bv2, Ws2, bs2)
  part2 = _fused_16(q2, k2, v2, src, dst)
  return _final_combine(part2, s2)
